# baseline (speedup 1.0000x reference)
import functools

import jax
import jax.numpy as jnp
from jax import lax
from jax.experimental import pallas as pl
from jax.experimental.pallas import tpu as pltpu
from jax.experimental.pallas import tpu_sc as plsc

_B, _T, _C = 2, 4096, 1024
_H = 4 * _C
_N = _B * _T

_BK1 = 1024
_NK1 = _N // _BK1
_SUB = 256
_NCH = _N // _SUB
_NVR = _SUB // 16
_MAXG = (_NVR + 2) // 3

_BT3 = 1024
_NT3 = _N // _BT3
_BH3 = 2048
_NH3 = _H // _BH3


def _k1_body(x_ref, wr_ref, selidx_ref, unsidx_ref, pstart_ref, ptot_ref, run_ref):
    i = pl.program_id(0)

    @pl.when(i == 0)
    def _():
        run_ref[0] = 0

    xb = x_ref[...]
    logits = lax.dot_general(xb, wr_ref[...], (((1,), (0,)), ((), ())),
                             preferred_element_type=jnp.float32)
    m = (logits > 0.0).astype(jnp.float32)

    colp = lax.broadcasted_iota(jnp.int32, (1, _SUB), 1).astype(jnp.float32)
    tokloc = colp
    lr = lax.broadcasted_iota(jnp.int32, (_SUB, _SUB), 0)
    lc = lax.broadcasted_iota(jnp.int32, (_SUB, _SUB), 1)
    l_strict = (lc < lr).astype(jnp.float32)

    run = run_ref[0]
    sel_rows, uns_rows, bases, cnts = [], [], [], []
    for s in range(_BK1 // _SUB):
        msub = lax.slice(m, (s * _SUB, 0), ((s + 1) * _SUB, 1))
        cnt = jnp.sum(msub).astype(jnp.int32)
        gbase = i * _BK1 + s * _SUB

        def compact(mcol, c):
            pos = lax.dot_general(l_strict, mcol, (((1,), (0,)), ((), ())),
                                  preferred_element_type=jnp.float32)
            ct = (pos == colp).astype(jnp.float32) * mcol
            row = lax.dot_general(tokloc, ct, (((1,), (0,)), ((), ())),
                                  preferred_element_type=jnp.float32)
            row = row + gbase
            return jnp.where(colp < c.astype(jnp.float32), row,
                             jnp.float32(_N)).astype(jnp.int32)

        sel_rows.append(compact(msub, cnt))
        uns_rows.append(compact(1.0 - msub, 256 - cnt))
        bases.append(run)
        cnts.append(cnt)
        run = run + ((cnt + 15) // 16) * 16
    run_ref[0] = run

    r3 = lax.broadcasted_iota(jnp.int32, (1, 4, _SUB), 1)
    sacc = jnp.zeros((1, 4, _SUB), jnp.int32)
    uacc = jnp.zeros((1, 4, _SUB), jnp.int32)
    for s in range(4):
        sacc = sacc + jnp.where(r3 == s, sel_rows[s][:, None, :], 0)
        uacc = uacc + jnp.where(r3 == s, uns_rows[s][:, None, :], 0)
    selidx_ref[...] = sacc
    unsidx_ref[...] = uacc

    r2 = lax.broadcasted_iota(jnp.int32, (1, 4, 16), 1)
    l2 = lax.broadcasted_iota(jnp.int32, (1, 4, 16), 2)
    acc = jnp.zeros((1, 4, 16), jnp.int32)
    for s in range(4):
        rowsel = (r2 == s).astype(jnp.int32)
        vals = jnp.where(l2 == 0, bases[s], jnp.where(l2 == 1, cnts[s], 0))
        acc = acc + rowsel * vals
    pstart_ref[...] = acc
    ptot_ref[...] = jnp.full((1, 16), run, jnp.int32)


def _k1(x2d, w_router):
    return pl.pallas_call(
        _k1_body,
        grid=(_NK1,),
        in_specs=[
            pl.BlockSpec((_BK1, _C), lambda i: (i, 0)),
            pl.BlockSpec((_C, 1), lambda i: (0, 0)),
        ],
        out_specs=[
            pl.BlockSpec((1, 4, _SUB), lambda i: (i, 0, 0)),
            pl.BlockSpec((1, 4, _SUB), lambda i: (i, 0, 0)),
            pl.BlockSpec((1, 4, 16), lambda i: (i, 0, 0)),
            pl.BlockSpec((1, 16), lambda i: (0, 0)),
        ],
        out_shape=[
            jax.ShapeDtypeStruct((_NK1, 4, _SUB), jnp.int32),
            jax.ShapeDtypeStruct((_NK1, 4, _SUB), jnp.int32),
            jax.ShapeDtypeStruct((_NK1, 4, 16), jnp.int32),
            jax.ShapeDtypeStruct((1, 16), jnp.int32),
        ],
        scratch_shapes=[pltpu.SMEM((1,), jnp.int32)],
        compiler_params=pltpu.CompilerParams(
            dimension_semantics=("arbitrary",),
        ),
    )(x2d, w_router)


_NBUF = 4


def _unit_pipe(u, mk_load, mk_store, bufs, lsems, ssems):
    nb_ = _NBUF
    d = nb_ // 2

    for p in range(min(d, _NVR)):
        @pl.when(p < u)
        def _(p=p):
            mk_load(p, bufs[p % nb_], lsems[p % nb_]).start()

    for i in range(_NVR):
        @pl.when(i < u)
        def _(i=i, b=i % nb_):
            mk_load(i, bufs[b], lsems[b]).wait()
            mk_store(i, bufs[b], ssems[b]).start()

        if i + d < _NVR:
            @pl.when(i + d < u)
            def _(i=i):
                if i - d >= 0:
                    mk_store(i - d, bufs[(i - d) % nb_], ssems[(i - d) % nb_]).wait()
                mk_load(i + d, bufs[(i + d) % nb_], lsems[(i + d) % nb_]).start()

    for k in range(_NVR):
        @pl.when((k < u) & (u <= k + 2 * d))
        def _(k=k, b=k % nb_):
            mk_store(k, bufs[b], ssems[b]).wait()


def _k2_body(selidx_hbm, x_hbm, pstart_hbm, xg_hbm,
             lidx, bufa, bufb, bufc, bufd, psv,
             gsa, gsb, gsc, gsd, ssa, ssb, ssc, ssd):
    w = lax.axis_index("s") * 2 + lax.axis_index("c")

    pltpu.sync_copy(selidx_hbm.at[w], lidx)
    pltpu.sync_copy(pstart_hbm.at[w], psv)
    pv = psv[pl.ds(0, 16)]
    base = pv[0]
    cnt = pv[1]
    u = (cnt + 15) // 16

    def load_k2(i, buf, sem):
        iv = jnp.minimum(lidx[pl.ds(i * 16, 16)], _N - 1)
        return pltpu.make_async_copy(x_hbm.at[iv], buf, sem)

    def store_k2(i, buf, sem):
        dst = pl.multiple_of(base + i * 16, 16)
        return pltpu.make_async_copy(buf, xg_hbm.at[pl.ds(dst, 16)], sem)

    _unit_pipe(u, load_k2, store_k2, [bufa, bufb, bufc, bufd],
               [gsa, gsb, gsc, gsd], [ssa, ssb, ssc, ssd])


def _k2(selidx, x2d, pstart):
    mesh = plsc.VectorSubcoreMesh(core_axis_name="c", subcore_axis_name="s")
    k = pl.kernel(
        _k2_body,
        out_type=jax.ShapeDtypeStruct((_N, _C), jnp.float32),
        mesh=mesh,
        scratch_types=[
            pltpu.VMEM((_SUB,), jnp.int32),
            pltpu.VMEM((16, _C), jnp.float32),
            pltpu.VMEM((16, _C), jnp.float32),
            pltpu.VMEM((16, _C), jnp.float32),
            pltpu.VMEM((16, _C), jnp.float32),
            pltpu.VMEM((16,), jnp.int32),
        ] + [pltpu.SemaphoreType.DMA] * 8,
    )
    return k(selidx, x2d, pstart)


def _k3_body(pt_ref, xg_ref, wfc_ref, wp_ref, o_ref, h_ref):
    i = pl.program_id(0)
    j = pl.program_id(1)
    active = jnp.maximum((pt_ref[0] + _BT3 - 1) // _BT3, 1)

    @pl.when(i < active)
    def _():
        @pl.when(j == 0)
        def _():
            xb = xg_ref[...]
            ms = jnp.mean(jnp.square(xb), axis=-1, keepdims=True)
            h_ref[...] = xb * lax.rsqrt(ms + 1e-6)
            o_ref[...] = jnp.zeros_like(o_ref)

        a = lax.dot_general(h_ref[...].astype(jnp.bfloat16), wfc_ref[...],
                            (((1,), (0,)), ((), ())),
                            preferred_element_type=jnp.float32)
        a = jnp.maximum(a, 0.0)
        a = (a * a).astype(jnp.bfloat16)
        o_ref[...] += lax.dot_general(a, wp_ref[...], (((1,), (0,)), ((), ())),
                                      preferred_element_type=jnp.float32)

        @pl.when(j == _NH3 - 1)
        def _():
            o_ref[...] = xg_ref[...] + o_ref[...]


def _last(pt_ref):
    return jnp.maximum((pt_ref[0] + _BT3 - 1) // _BT3, 1) - 1


def _k3(ptot, xg, w_fc, w_proj):
    grid_spec = pltpu.PrefetchScalarGridSpec(
        num_scalar_prefetch=1,
        grid=(_NT3, _NH3),
        in_specs=[
            pl.BlockSpec((_BT3, _C), lambda i, j, pt: (jnp.minimum(i, _last(pt)), 0)),
            pl.BlockSpec((_C, _BH3),
                         lambda i, j, pt: (0, jnp.where(i <= _last(pt), j, _NH3 - 1))),
            pl.BlockSpec((_BH3, _C),
                         lambda i, j, pt: (jnp.where(i <= _last(pt), j, _NH3 - 1), 0)),
        ],
        out_specs=pl.BlockSpec((_BT3, _C), lambda i, j, pt: (jnp.minimum(i, _last(pt)), 0)),
        scratch_shapes=[pltpu.VMEM((_BT3, _C), jnp.float32)],
    )
    return pl.pallas_call(
        _k3_body,
        grid_spec=grid_spec,
        out_shape=jax.ShapeDtypeStruct((_N, _C), jnp.float32),
        compiler_params=pltpu.CompilerParams(
            dimension_semantics=("arbitrary", "arbitrary"),
        ),
    )(ptot, xg, w_fc, w_proj)


def _k4_body(selidx_hbm, unsidx_hbm, x_hbm, pstart_hbm, yg_hbm, out_hbm,
             s1d, u1d, sscat, uscat, bufa, bufb, bufc, bufd, psv,
             gsa, gsb, gsc, gsd, ssa, ssb, ssc, ssd):
    w = lax.axis_index("s") * 2 + lax.axis_index("c")

    pltpu.sync_copy(selidx_hbm.at[w], s1d)
    pltpu.sync_copy(unsidx_hbm.at[w], u1d)
    pltpu.sync_copy(pstart_hbm.at[w], psv)
    pv = psv[pl.ds(0, 16)]
    base = pv[0]
    cs = pv[1]
    cu = _SUB - cs

    for t in range(_NVR):
        sscat[t] = s1d[pl.ds(t * 16, 16)]
        uscat[t] = u1d[pl.ds(t * 16, 16)]

    bufs = [bufa, bufb, bufc, bufd]
    lsems = [gsa, gsb, gsc, gsd]
    ssems = [ssa, ssb, ssc, ssd]

    def load_sel(i, buf, sem):
        src = pl.multiple_of(base + i * 16, 16)
        return pltpu.make_async_copy(yg_hbm.at[pl.ds(src, 16)], buf, sem)

    def store_sel(i, buf, sem):
        return pltpu.make_async_copy(buf, out_hbm.at[sscat.at[i]], sem)

    _unit_pipe((cs + 15) // 16, load_sel, store_sel, bufs, lsems, ssems)

    def load_uns(i, buf, sem):
        iv = jnp.minimum(uscat[i], _N - 1)
        return pltpu.make_async_copy(x_hbm.at[iv], buf, sem)

    def store_uns(i, buf, sem):
        return pltpu.make_async_copy(buf, out_hbm.at[uscat.at[i]], sem)

    _unit_pipe((cu + 15) // 16, load_uns, store_uns, bufs, lsems, ssems)


def _k4(selidx, unsidx, x2d, pstart, yg):
    mesh = plsc.VectorSubcoreMesh(core_axis_name="c", subcore_axis_name="s")
    k = pl.kernel(
        _k4_body,
        out_type=jax.ShapeDtypeStruct((_N + 16, _C), jnp.float32),
        mesh=mesh,
        scratch_types=[
            pltpu.VMEM((_SUB,), jnp.int32),
            pltpu.VMEM((_SUB,), jnp.int32),
            pltpu.VMEM((_NVR, 16), jnp.int32),
            pltpu.VMEM((_NVR, 16), jnp.int32),
            pltpu.VMEM((16, _C), jnp.float32),
            pltpu.VMEM((16, _C), jnp.float32),
            pltpu.VMEM((16, _C), jnp.float32),
            pltpu.VMEM((16, _C), jnp.float32),
            pltpu.VMEM((16,), jnp.int32),
        ] + [pltpu.SemaphoreType.DMA] * 8,
    )
    return k(selidx, unsidx, x2d, pstart, yg)


def kernel(x, w_router, w_fc, w_proj):
    x2d = x.reshape(_N, _C)
    selidx, unsidx, pstart, ptot = _k1(x2d, w_router)
    sel = selidx.reshape(_NCH, _SUB)
    uns = unsidx.reshape(_NCH, _SUB)
    pst = pstart.reshape(_NCH, 16)
    xg = _k2(sel, x2d, pst)
    yg = _k3(ptot.reshape(16), xg,
             w_fc.astype(jnp.bfloat16), w_proj.astype(jnp.bfloat16))
    outp = _k4(sel, uns, x2d, pst, yg)
    return outp[:_N].reshape(_B, _T, _C)


# speedup vs baseline: 1.7075x; 1.7075x over previous
import functools

import jax
import jax.numpy as jnp
from jax.experimental import pallas as pl
from jax.experimental.pallas import tpu as pltpu

_B, _T, _C = 2, 4096, 1024
_H = 4 * _C
_N = _B * _T
_BT = 1024
_BH = 1024
_NT = _N // _BT
_NH = _H // _BH


def _dense_body(x_ref, wr_ref, wfc_ref, wp_ref, o_ref, h_ref):
    j = pl.program_id(1)

    @pl.when(j == 0)
    def _():
        xb = x_ref[...]
        ms = jnp.mean(jnp.square(xb), axis=-1, keepdims=True)
        h_ref[...] = xb * jax.lax.rsqrt(ms + 1e-6)
        o_ref[...] = jnp.zeros_like(o_ref)

    a = jax.lax.dot_general(h_ref[...], wfc_ref[...], (((1,), (0,)), ((), ())),
                            preferred_element_type=jnp.float32)
    a = jnp.maximum(a, 0.0)
    a = a * a
    o_ref[...] += jax.lax.dot_general(a, wp_ref[...], (((1,), (0,)), ((), ())),
                                      preferred_element_type=jnp.float32)

    @pl.when(j == _NH - 1)
    def _():
        xb = x_ref[...]
        logits = jax.lax.dot_general(xb, wr_ref[...], (((1,), (0,)), ((), ())),
                                     preferred_element_type=jnp.float32)
        mask = (logits > 0.0).astype(jnp.float32)
        o_ref[...] = xb + o_ref[...] * mask


@functools.partial(jax.jit, static_argnums=())
def _dense(x2d, w_router, w_fc, w_proj):
    return pl.pallas_call(
        _dense_body,
        grid=(_NT, _NH),
        in_specs=[
            pl.BlockSpec((_BT, _C), lambda i, j: (i, 0)),
            pl.BlockSpec((_C, 1), lambda i, j: (0, 0)),
            pl.BlockSpec((_C, _BH), lambda i, j: (0, j)),
            pl.BlockSpec((_BH, _C), lambda i, j: (j, 0)),
        ],
        out_specs=pl.BlockSpec((_BT, _C), lambda i, j: (i, 0)),
        out_shape=jax.ShapeDtypeStruct((_N, _C), jnp.float32),
        scratch_shapes=[pltpu.VMEM((_BT, _C), jnp.float32)],
        compiler_params=pltpu.CompilerParams(
            dimension_semantics=("arbitrary", "arbitrary"),
        ),
    )(x2d, w_router, w_fc, w_proj)


def kernel(x, w_router, w_fc, w_proj):
    x2d = x.reshape(_N, _C)
    out = _dense(x2d, w_router, w_fc, w_proj)
    return out.reshape(_B, _T, _C)
